# Initial kernel scaffold; baseline (speedup 1.0000x reference)
import jax
import jax.numpy as jnp
from jax.experimental import pallas as pl


def kernel(x, table):
    raise NotImplementedError("write your pallas kernel here")


# speedup vs baseline: 472.7898x; 472.7898x over previous
import functools

import jax
import jax.numpy as jnp
import numpy as np
from jax import lax
from jax.experimental import pallas as pl
from jax.experimental.pallas import tpu as pltpu
from jax.experimental.pallas import tpu_sc as plsc

NC = 2
NS = 16
NW = NC * NS
L = 16

MAGIC = np.float32(1.5 * 2**23)
TABLE_N = 256
CHUNK = 32768


def _secgelu_body(x_hbm, table_hbm, out_hbm, table_v, buf, n_chunks):
    wid = lax.axis_index("s") * NC + lax.axis_index("c")
    per_w = n_chunks * CHUNK
    base = wid * per_w
    pltpu.sync_copy(table_hbm, table_v)

    @pl.loop(0, n_chunks)
    def _chunk(j):
        off = pl.multiple_of(base + j * CHUNK, CHUNK)
        pltpu.sync_copy(x_hbm.at[pl.ds(off, CHUNK)], buf)

        @pl.loop(0, CHUNK // L, unroll=8)
        def _vec(i):
            s = pl.ds(i * L, L)
            xv = buf[s]
            t = xv * np.float32(2.0**6)
            r = (t + MAGIC) - MAGIC
            a = jnp.abs(r)
            c = jnp.minimum(a, np.float32(TABLE_N - 1))
            tv = plsc.load_gather(table_v, [c.astype(jnp.int32)])
            relu = jnp.where(r >= 0.0, xv, np.float32(0.0))
            buf[s] = relu - tv

        pltpu.sync_copy(buf, out_hbm.at[pl.ds(off, CHUNK)])


def kernel(x, table):
    n = x.size
    assert n % (NW * CHUNK) == 0
    n_chunks = n // (NW * CHUNK)
    mesh = plsc.VectorSubcoreMesh(
        core_axis_name="c", subcore_axis_name="s",
        num_cores=NC, num_subcores=NS)
    body = functools.partial(_secgelu_body, n_chunks=n_chunks)
    flat = pl.kernel(
        body,
        out_type=jax.ShapeDtypeStruct((n,), jnp.float32),
        mesh=mesh,
        scratch_types=[
            pltpu.VMEM((TABLE_N,), jnp.float32),
            pltpu.VMEM((CHUNK,), jnp.float32),
        ],
        compiler_params=pltpu.CompilerParams(needs_layout_passes=False),
        name="secgelu_sc",
    )(x.reshape(n), table)
    return flat.reshape(x.shape)


# speedup vs baseline: 552.3984x; 1.1684x over previous
import functools

import jax
import jax.numpy as jnp
import numpy as np
from jax import lax
from jax.experimental import pallas as pl
from jax.experimental.pallas import tpu as pltpu
from jax.experimental.pallas import tpu_sc as plsc

NC = 2
NS = 16
NW = NC * NS
L = 16

MAGIC = np.float32(1.5 * 2**23)
TABLE_N = 256
CHUNK = 16384
NB = 4


def _compute_chunk(buf, table_v):
    @pl.loop(0, CHUNK // L, unroll=8)
    def _vec(i):
        s = pl.ds(i * L, L)
        xv = buf[s]
        t = xv * np.float32(2.0**6)
        r = (t + MAGIC) - MAGIC
        a = jnp.abs(r)
        c = jnp.minimum(a, np.float32(TABLE_N - 1))
        tv = plsc.load_gather(table_v, [c.astype(jnp.int32)])
        relu = jnp.where(r >= 0.0, xv, np.float32(0.0))
        buf[s] = relu - tv


def _secgelu_body(x_hbm, table_hbm, out_hbm, table_v, b0, b1, b2, b3,
                  isem, osem, n_chunks):
    bufs = [b0, b1, b2, b3]
    wid = lax.axis_index("s") * NC + lax.axis_index("c")
    base = wid * (n_chunks * CHUNK)
    pltpu.sync_copy(table_hbm, table_v)

    def src(j):
        return x_hbm.at[pl.ds(pl.multiple_of(base + j * CHUNK, CHUNK), CHUNK)]

    def dst(j):
        return out_hbm.at[pl.ds(pl.multiple_of(base + j * CHUNK, CHUNK), CHUNK)]

    pltpu.async_copy(src(0), bufs[0], isem.at[0])
    pltpu.async_copy(src(1), bufs[1], isem.at[1])

    @pl.loop(0, n_chunks, step=NB)
    def _ring(j):
        for b in range(NB):
            jj = j + b
            pltpu.make_async_copy(src(jj), bufs[b], isem.at[b]).wait()
            _compute_chunk(bufs[b], table_v)
            pltpu.async_copy(bufs[b], dst(jj), osem.at[b])

            bp = (b + 2) % NB

            @pl.when(jj + 2 < n_chunks)
            def _prefetch():
                @pl.when(jj >= NB - 2)
                def _reclaim():
                    pltpu.make_async_copy(
                        bufs[bp], dst(jj + 2 - NB), osem.at[bp]).wait()

                pltpu.async_copy(src(jj + 2), bufs[bp], isem.at[bp])

    for b in range(NB):
        jj = n_chunks - NB + b
        pltpu.make_async_copy(bufs[b], dst(jj), osem.at[b]).wait()


def kernel(x, table):
    n = x.size
    assert n % (NW * CHUNK) == 0
    n_chunks = n // (NW * CHUNK)
    mesh = plsc.VectorSubcoreMesh(
        core_axis_name="c", subcore_axis_name="s",
        num_cores=NC, num_subcores=NS)
    body = functools.partial(_secgelu_body, n_chunks=n_chunks)
    flat = pl.kernel(
        body,
        out_type=jax.ShapeDtypeStruct((n,), jnp.float32),
        mesh=mesh,
        scratch_types=[
            pltpu.VMEM((TABLE_N,), jnp.float32),
            pltpu.VMEM((CHUNK,), jnp.float32),
            pltpu.VMEM((CHUNK,), jnp.float32),
            pltpu.VMEM((CHUNK,), jnp.float32),
            pltpu.VMEM((CHUNK,), jnp.float32),
            pltpu.SemaphoreType.DMA((NB,)),
            pltpu.SemaphoreType.DMA((NB,)),
        ],
        compiler_params=pltpu.CompilerParams(needs_layout_passes=False),
        name="secgelu_sc",
    )(x.reshape(n), table)
    return flat.reshape(x.shape)


# speedup vs baseline: 942.0187x; 1.7053x over previous
import functools

import jax
import jax.numpy as jnp
import numpy as np
from jax import lax
from jax.experimental import pallas as pl
from jax.experimental.pallas import tpu as pltpu
from jax.experimental.pallas import tpu_sc as plsc

NC = 2
NS = 16
NW = NC * NS
L = 16

MAGIC = np.float32(1.5 * 2**23)
TABLE_N = 256
CHUNK = 16384
NB = 4


def _compute_chunk(buf, table_v):
    @plsc.parallel_loop(0, CHUNK // L, unroll=8)
    def _vec(i):
        s = pl.ds(i * L, L)
        xv = buf[s]
        t = xv * np.float32(2.0**6)
        r = (t + MAGIC) - MAGIC
        a = jnp.abs(r)
        c = jnp.minimum(a, np.float32(TABLE_N - 1))
        tv = plsc.load_gather(table_v, [c.astype(jnp.int32)])
        relu = jnp.where(r >= 0.0, xv, np.float32(0.0))
        buf[s] = relu - tv


def _secgelu_body(x_hbm, table_hbm, out_hbm, table_v, b0, b1, b2, b3,
                  isem, osem, n_chunks):
    bufs = [b0, b1, b2, b3]
    wid = lax.axis_index("s") * NC + lax.axis_index("c")
    base = wid * (n_chunks * CHUNK)
    pltpu.sync_copy(table_hbm, table_v)

    def src(j):
        return x_hbm.at[pl.ds(pl.multiple_of(base + j * CHUNK, CHUNK), CHUNK)]

    def dst(j):
        return out_hbm.at[pl.ds(pl.multiple_of(base + j * CHUNK, CHUNK), CHUNK)]

    pltpu.async_copy(src(0), bufs[0], isem.at[0])
    pltpu.async_copy(src(1), bufs[1], isem.at[1])

    @pl.loop(0, n_chunks, step=NB)
    def _ring(j):
        for b in range(NB):
            jj = j + b
            pltpu.make_async_copy(src(jj), bufs[b], isem.at[b]).wait()
            _compute_chunk(bufs[b], table_v)
            pltpu.async_copy(bufs[b], dst(jj), osem.at[b])

            bp = (b + 2) % NB

            @pl.when(jj + 2 < n_chunks)
            def _prefetch():
                @pl.when(jj >= NB - 2)
                def _reclaim():
                    pltpu.make_async_copy(
                        bufs[bp], dst(jj + 2 - NB), osem.at[bp]).wait()

                pltpu.async_copy(src(jj + 2), bufs[bp], isem.at[bp])

    for b in range(NB):
        jj = n_chunks - NB + b
        pltpu.make_async_copy(bufs[b], dst(jj), osem.at[b]).wait()


def kernel(x, table):
    n = x.size
    assert n % (NW * CHUNK) == 0
    n_chunks = n // (NW * CHUNK)
    mesh = plsc.VectorSubcoreMesh(
        core_axis_name="c", subcore_axis_name="s",
        num_cores=NC, num_subcores=NS)
    body = functools.partial(_secgelu_body, n_chunks=n_chunks)
    flat = pl.kernel(
        body,
        out_type=jax.ShapeDtypeStruct((n,), jnp.float32),
        mesh=mesh,
        scratch_types=[
            pltpu.VMEM((TABLE_N,), jnp.float32),
            pltpu.VMEM((CHUNK,), jnp.float32),
            pltpu.VMEM((CHUNK,), jnp.float32),
            pltpu.VMEM((CHUNK,), jnp.float32),
            pltpu.VMEM((CHUNK,), jnp.float32),
            pltpu.SemaphoreType.DMA((NB,)),
            pltpu.SemaphoreType.DMA((NB,)),
        ],
        compiler_params=pltpu.CompilerParams(needs_layout_passes=False),
        name="secgelu_sc",
    )(x.reshape(n), table)
    return flat.reshape(x.shape)


# speedup vs baseline: 1937.3840x; 2.0566x over previous
import functools

import jax
import jax.numpy as jnp
import numpy as np
from jax import lax
from jax.experimental import pallas as pl
from jax.experimental.pallas import tpu as pltpu
from jax.experimental.pallas import tpu_sc as plsc

NC = 2
NS = 16
NW = NC * NS
L = 16

MAGIC = np.float32(1.5 * 2**23)
TABLE_N = 256
ROWS = 8
COLS = 2048
NB = 4


def _compute_slab(buf, table_v):
    @plsc.parallel_loop(0, COLS // L)
    def _vec(i):
        s = pl.ds(i * L, L)
        for r in range(ROWS):
            xv = buf[r, s]
            t = xv * np.float32(2.0**6)
            u = t + MAGIC
            r_ = u - MAGIC
            a = jnp.abs(r_)
            c = jnp.minimum(a, np.float32(TABLE_N - 1))
            tv = plsc.load_gather(table_v, [c.astype(jnp.int32)])
            relu = jnp.where(u >= MAGIC, xv, np.float32(0.0))
            buf[r, s] = relu - tv


def _secgelu_body(x_hbm, table_hbm, out_hbm, table_v, b0, b1, b2, b3,
                  isem, osem, n_chunks):
    bufs = [b0, b1, b2, b3]
    wid = lax.axis_index("s") * NC + lax.axis_index("c")
    batch = wid // 16
    row0 = (wid % 16) * 512
    pltpu.sync_copy(table_hbm, table_v)

    def slab(ref, jj, b):
        row = pl.multiple_of(row0 + (jj >> 1) * ROWS, ROWS)
        return ref.at[batch, pl.ds(row, ROWS), pl.ds((b % 2) * COLS, COLS)]

    pltpu.async_copy(slab(x_hbm, 0, 0), bufs[0], isem.at[0])
    pltpu.async_copy(slab(x_hbm, 1, 1), bufs[1], isem.at[1])

    @pl.loop(0, n_chunks, step=NB)
    def _ring(j):
        for b in range(NB):
            jj = j + b
            pltpu.make_async_copy(slab(x_hbm, jj, b), bufs[b],
                                  isem.at[b]).wait()
            _compute_slab(bufs[b], table_v)
            pltpu.async_copy(bufs[b], slab(out_hbm, jj, b), osem.at[b])

            bp = (b + 2) % NB

            @pl.when(jj + 2 < n_chunks)
            def _prefetch():
                @pl.when(jj >= NB - 2)
                def _reclaim():
                    pltpu.make_async_copy(
                        bufs[bp], slab(out_hbm, jj + 2 - NB, bp),
                        osem.at[bp]).wait()

                pltpu.async_copy(slab(x_hbm, jj + 2, bp), bufs[bp],
                                 isem.at[bp])

    for b in range(NB):
        jj = n_chunks - NB + b
        pltpu.make_async_copy(bufs[b], slab(out_hbm, jj, b),
                              osem.at[b]).wait()


def kernel(x, table):
    assert x.shape == (2, 8192, 4096) and x.dtype == jnp.float32
    n_chunks = (8192 // 16) // ROWS * 2
    mesh = plsc.VectorSubcoreMesh(
        core_axis_name="c", subcore_axis_name="s",
        num_cores=NC, num_subcores=NS)
    body = functools.partial(_secgelu_body, n_chunks=n_chunks)
    return pl.kernel(
        body,
        out_type=jax.ShapeDtypeStruct(x.shape, jnp.float32),
        mesh=mesh,
        scratch_types=[
            pltpu.VMEM((TABLE_N,), jnp.float32),
            pltpu.VMEM((ROWS, COLS), jnp.float32),
            pltpu.VMEM((ROWS, COLS), jnp.float32),
            pltpu.VMEM((ROWS, COLS), jnp.float32),
            pltpu.VMEM((ROWS, COLS), jnp.float32),
            pltpu.SemaphoreType.DMA((NB,)),
            pltpu.SemaphoreType.DMA((NB,)),
        ],
        compiler_params=pltpu.CompilerParams(needs_layout_passes=False),
        name="secgelu_sc",
    )(x, table)


# speedup vs baseline: 2809.5867x; 1.4502x over previous
import functools

import jax
import jax.numpy as jnp
import numpy as np
from jax import lax
from jax.experimental import pallas as pl
from jax.experimental.pallas import tpu as pltpu
from jax.experimental.pallas import tpu_sc as plsc

NC = 2
NS = 16
NW = NC * NS
L = 16

MAGIC = np.float32(1.5 * 2**23)
TABLE_N = 256
ROWS = 8
COLS = 2048
NB = 4


CENTER = TABLE_N * 2 // 2
JBIAS = np.int32(0x4B400000 - 256)


def _build_sym_table(table_v, t2_v):
    @pl.loop(0, 2 * TABLE_N // L)
    def _b(i):
        k = i * L + lax.iota(jnp.int32, L)
        d = k - np.int32(CENTER)
        a = jnp.minimum(jnp.abs(d), np.int32(TABLE_N - 1))
        t2_v[pl.ds(i * L, L)] = plsc.load_gather(table_v, [a])


def _compute_slab(buf, t2_v):
    @plsc.parallel_loop(0, COLS // L)
    def _vec(i):
        s = pl.ds(i * L, L)
        for r in range(ROWS):
            xv = buf[r, s]
            u = xv * np.float32(2.0**6) + MAGIC
            j = plsc.bitcast(u, jnp.int32) - JBIAS
            idx = jnp.minimum(plsc.bitcast(j, jnp.uint32),
                              np.uint32(2 * TABLE_N - 1))
            tv = plsc.load_gather(t2_v, [plsc.bitcast(idx, jnp.int32)])
            relu = jnp.where(u >= MAGIC, xv, np.float32(0.0))
            buf[r, s] = relu - tv


def _secgelu_body(x_hbm, table_hbm, out_hbm, table_v, t2_v, b0, b1, b2, b3,
                  isem, osem, n_chunks):
    bufs = [b0, b1, b2, b3]
    wid = lax.axis_index("s") * NC + lax.axis_index("c")
    batch = wid // 16
    row0 = (wid % 16) * 512
    pltpu.sync_copy(table_hbm, table_v)
    _build_sym_table(table_v, t2_v)

    def slab(ref, jj, b):
        row = pl.multiple_of(row0 + (jj >> 1) * ROWS, ROWS)
        return ref.at[batch, pl.ds(row, ROWS), pl.ds((b % 2) * COLS, COLS)]

    pltpu.async_copy(slab(x_hbm, 0, 0), bufs[0], isem.at[0])
    pltpu.async_copy(slab(x_hbm, 1, 1), bufs[1], isem.at[1])

    @pl.loop(0, n_chunks, step=NB)
    def _ring(j):
        for b in range(NB):
            jj = j + b
            pltpu.make_async_copy(slab(x_hbm, jj, b), bufs[b],
                                  isem.at[b]).wait()
            _compute_slab(bufs[b], t2_v)
            pltpu.async_copy(bufs[b], slab(out_hbm, jj, b), osem.at[b])

            bp = (b + 2) % NB

            @pl.when(jj + 2 < n_chunks)
            def _prefetch():
                @pl.when(jj >= NB - 2)
                def _reclaim():
                    pltpu.make_async_copy(
                        bufs[bp], slab(out_hbm, jj + 2 - NB, bp),
                        osem.at[bp]).wait()

                pltpu.async_copy(slab(x_hbm, jj + 2, bp), bufs[bp],
                                 isem.at[bp])

    for b in range(NB):
        jj = n_chunks - NB + b
        pltpu.make_async_copy(bufs[b], slab(out_hbm, jj, b),
                              osem.at[b]).wait()


def kernel(x, table):
    assert x.shape == (2, 8192, 4096) and x.dtype == jnp.float32
    n_chunks = (8192 // 16) // ROWS * 2
    mesh = plsc.VectorSubcoreMesh(
        core_axis_name="c", subcore_axis_name="s",
        num_cores=NC, num_subcores=NS)
    body = functools.partial(_secgelu_body, n_chunks=n_chunks)
    return pl.kernel(
        body,
        out_type=jax.ShapeDtypeStruct(x.shape, jnp.float32),
        mesh=mesh,
        scratch_types=[
            pltpu.VMEM((TABLE_N,), jnp.float32),
            pltpu.VMEM((2 * TABLE_N,), jnp.float32),
            pltpu.VMEM((ROWS, COLS), jnp.float32),
            pltpu.VMEM((ROWS, COLS), jnp.float32),
            pltpu.VMEM((ROWS, COLS), jnp.float32),
            pltpu.VMEM((ROWS, COLS), jnp.float32),
            pltpu.SemaphoreType.DMA((NB,)),
            pltpu.SemaphoreType.DMA((NB,)),
        ],
        compiler_params=pltpu.CompilerParams(needs_layout_passes=False),
        name="secgelu_sc",
    )(x, table)
